# baseline (speedup 1.0000x reference)
import functools

import jax
import jax.numpy as jnp
from jax import lax
from jax.experimental import pallas as pl
from jax.experimental.pallas import tpu as pltpu
from jax.experimental.pallas import tpu_sc as plsc

N = 10000
E = 320000
D_IN = 128
HID = 32
HEADS = 8
B = 16

CH = 128
NTILES = 32
EP = 331776
PAD = EP - (E + N)
NSEGP = 10112
SEG_PER_TILE = NSEGP // 16

_f32 = jnp.float32


def _dense1_body(x_ref, wt_ref, bt_ref, wl_ref, bl_ref, wr_ref, br_ref,
                 xl_ref, xr_ref):
    h0 = jnp.dot(x_ref[...], wt_ref[...], preferred_element_type=_f32) + bt_ref[...]
    xl_ref[...] = jnp.dot(h0, wl_ref[...], preferred_element_type=_f32) + bl_ref[...]
    xr_ref[...] = jnp.dot(h0, wr_ref[...], preferred_element_type=_f32) + br_ref[...]


def _dense1(x, Wt, bt, Wl, bl, Wr, br):
    K = Wl.shape[1]
    return pl.pallas_call(
        _dense1_body,
        out_shape=(jax.ShapeDtypeStruct((N, K), _f32),
                   jax.ShapeDtypeStruct((N, K), _f32)),
    )(x, Wt, bt.reshape(1, -1), Wl, bl.reshape(1, -1), Wr, br.reshape(1, -1))


def _edge1_body(xl_ref, xr_ref, a_ref, b_ref, ee_ref, w_ref):
    xl = xl_ref[...].astype(_f32)
    m = xl + xr_ref[...].astype(_f32)
    m = jnp.where(m > 0, m, 0.2 * m)
    ee = jnp.exp(jnp.dot(m, a_ref[...], preferred_element_type=_f32))
    ee_ref[...] = jnp.concatenate(
        [ee, jnp.zeros((ee.shape[0], 8), _f32)], axis=1)
    af = jnp.dot(ee, b_ref[...], preferred_element_type=_f32)
    w_ref[...] = af * xl


def _edge1(xls, xrd, a, bexp):
    D = xls.shape[1]
    blk = 4096
    return pl.pallas_call(
        _edge1_body,
        grid=(EP // blk,),
        in_specs=[pl.BlockSpec((blk, D), lambda i: (i, 0)),
                  pl.BlockSpec((blk, D), lambda i: (i, 0)),
                  pl.BlockSpec((D, 8), lambda i: (0, 0)),
                  pl.BlockSpec((8, D), lambda i: (0, 0))],
        out_specs=(pl.BlockSpec((blk, 16), lambda i: (i, 0)),
                   pl.BlockSpec((blk, D), lambda i: (i, 0))),
        out_shape=(jax.ShapeDtypeStruct((EP, 16), _f32),
                   jax.ShapeDtypeStruct((EP, D), _f32)),
    )(xls, xrd, a, bexp)


def _edge4_body(xl_ref, xr_ref, a_ref, b_ref, w_ref):
    m = xl_ref[...] + xr_ref[...]
    m = jnp.where(m > 0, m, 0.2 * m)
    ee = jnp.exp(jnp.dot(m, a_ref[...], preferred_element_type=_f32))
    af = jnp.dot(ee, b_ref[...], preferred_element_type=_f32)
    w = af * xl_ref[...]
    w_ref[...] = jnp.concatenate(
        [w, ee[:, :1], jnp.zeros((w.shape[0], 7), _f32)], axis=1)


def _edge4(xls, xrd, a, bexp):
    blk = 4096
    return pl.pallas_call(
        _edge4_body,
        grid=(EP // blk,),
        in_specs=[pl.BlockSpec((blk, HID), lambda i: (i, 0)),
                  pl.BlockSpec((blk, HID), lambda i: (i, 0)),
                  pl.BlockSpec((HID, 8), lambda i: (0, 0)),
                  pl.BlockSpec((8, HID), lambda i: (0, 0))],
        out_specs=pl.BlockSpec((blk, HID + 8), lambda i: (i, 0)),
        out_shape=jax.ShapeDtypeStruct((EP, HID + 8), _f32),
    )(xls, xrd, a, bexp)


def _gn1_body(u_ref, d0_ref, d1_ref, be_ref, b1_ref, m_ref, mt_ref, ms_ref,
              w_ref, bb_ref, wl_ref, bl_ref, wr_ref, br_ref,
              h1_ref, xl_ref, xr_ref):
    den8 = d0_ref[...][:, :8] + d1_ref[...][:, :8]
    den = jnp.dot(den8, be_ref[...], preferred_element_type=_f32)
    x = u_ref[...] / (den + 1e-16) + b1_ref[...]
    M = m_ref[...]
    Mt = mt_ref[...]
    cnt = jnp.maximum(jnp.sum(M, axis=1, keepdims=True), 1.0)
    mean = jnp.dot(M, x, preferred_element_type=_f32) / cnt
    xc = x - jnp.dot(Mt, mean, preferred_element_type=_f32) * ms_ref[...]
    var = jnp.dot(M, xc * xc, preferred_element_type=_f32) / cnt
    rstd = lax.rsqrt(var + 1e-5)
    y = w_ref[...] * xc * jnp.dot(Mt, rstd, preferred_element_type=_f32) + bb_ref[...]
    h = jnp.where(y > 0, y, jnp.exp(jnp.minimum(y, 0.0)) - 1.0)
    h1_ref[...] = h
    xl_ref[...] = jnp.dot(h, wl_ref[...], preferred_element_type=_f32) + bl_ref[...]
    xr_ref[...] = jnp.dot(h, wr_ref[...], preferred_element_type=_f32) + br_ref[...]


def _gn1(u, d0, d1, bexp, bias1, M, Mt, ms, w, b, Wl4, bl4, Wr4, br4):
    return pl.pallas_call(
        _gn1_body,
        out_shape=(jax.ShapeDtypeStruct((N, HEADS * HID), _f32),
                   jax.ShapeDtypeStruct((N, HID), _f32),
                   jax.ShapeDtypeStruct((N, HID), _f32)),
    )(u, d0, d1, bexp, bias1.reshape(1, -1), M, Mt, ms.reshape(1, -1),
      w.reshape(1, -1), b.reshape(1, -1), Wl4, bl4.reshape(1, -1),
      Wr4, br4.reshape(1, -1))


def _final_body(ua_ref, ub_ref, b4_ref, m_ref, mt_ref,
                ms_ref, w_ref, bb_ref, h1_ref, s_ref, wl1_ref, wl2_ref,
                blin_ref, out_ref):
    upacked = ua_ref[...] + ub_ref[...]
    den1 = upacked[:, HID:HID + 1]
    u = upacked[:, :HID]
    x = u / (den1 + 1e-16) + b4_ref[...]
    M = m_ref[...]
    Mt = mt_ref[...]
    cnt = jnp.maximum(jnp.sum(M, axis=1, keepdims=True), 1.0)
    mean = jnp.dot(M, x, preferred_element_type=_f32) / cnt
    xc = x - jnp.dot(Mt, mean, preferred_element_type=_f32) * ms_ref[...]
    var = jnp.dot(M, xc * xc, preferred_element_type=_f32) / cnt
    rstd = lax.rsqrt(var + 1e-5)
    h2 = w_ref[...] * xc * jnp.dot(Mt, rstd, preferred_element_type=_f32) + bb_ref[...]
    S = s_ref[...]
    s1 = jnp.dot(S, h1_ref[...], preferred_element_type=_f32)
    s2 = jnp.dot(S, h2, preferred_element_type=_f32)
    out = (jnp.sum(s1 * wl1_ref[...], axis=1, keepdims=True)
           + jnp.sum(s2 * wl2_ref[...], axis=1, keepdims=True)
           + blin_ref[...])
    out_ref[...] = out


def _final(u0, u1, bias4, M, Mt, ms, w, b, h1, S, Wlin, blin):
    wl1 = Wlin[:HEADS * HID].reshape(1, -1)
    wl2 = Wlin[HEADS * HID:].reshape(1, -1)
    return pl.pallas_call(
        _final_body,
        out_shape=jax.ShapeDtypeStruct((B, 1), _f32),
    )(u0, u1, bias4.reshape(1, -1), M, Mt, ms.reshape(1, -1),
      w.reshape(1, -1), b.reshape(1, -1), h1, S, wl1, wl2, blin.reshape(1, 1))


@functools.cache
def _sc_mesh():
    return plsc.VectorSubcoreMesh(core_axis_name="c", subcore_axis_name="s")


@functools.cache
def _sc_gather(n_table, D, dtype=jnp.float32, tiled=True):
    per_w = EP // (NTILES * CH)

    @functools.partial(
        pl.kernel,
        out_type=jax.ShapeDtypeStruct((EP, D), dtype),
        mesh=_sc_mesh(),
        scratch_types=[pltpu.VMEM((1, CH), jnp.int32),
                       pltpu.VMEM((CH, D), dtype),
                       pltpu.SemaphoreType.DMA],
        compiler_params=pltpu.CompilerParams(use_tc_tiling_on_sc=tiled),
    )
    def k(table_hbm, idx_hbm, out_hbm, idx_v, rows_v, sem):
        c = lax.axis_index("c")
        s = lax.axis_index("s")
        wid = s * 2 + c

        def body(j, carry):
            base = pl.multiple_of((wid * per_w + j) * CH, CH)
            pltpu.sync_copy(idx_hbm.at[pl.ds(base, CH)], idx_v.at[0])
            pltpu.async_copy(table_hbm.at[idx_v.at[0]], rows_v, sem).wait()
            pltpu.sync_copy(rows_v, out_hbm.at[pl.ds(base, CH)])
            return carry

        lax.fori_loop(0, per_w, body, 0)

    return k


@functools.cache
def _sc_scatter_feat(Dh):
    per_w = EP // (16 * CH)

    @functools.partial(
        pl.kernel,
        out_type=jax.ShapeDtypeStruct((NSEGP, 2 * Dh), _f32),
        mesh=_sc_mesh(),
        scratch_types=[pltpu.VMEM((1, CH), jnp.int32),
                       pltpu.VMEM((CH, Dh), _f32),
                       pltpu.VMEM_SHARED((NSEGP, Dh), _f32)],
    )
    def k(rows_hbm, idx_hbm, zer_hbm, out_hbm, idx_v, rows_v, acc):
        c = lax.axis_index("c")
        s = lax.axis_index("s")
        seg0 = pl.multiple_of(s * SEG_PER_TILE, 8)
        pltpu.sync_copy(zer_hbm.at[pl.ds(seg0, SEG_PER_TILE)],
                        acc.at[pl.ds(seg0, SEG_PER_TILE)])
        plsc.subcore_barrier()

        def run(cc):
            off = cc * Dh

            def body(j, carry):
                base = pl.multiple_of((s * per_w + j) * CH, CH)
                pltpu.sync_copy(idx_hbm.at[pl.ds(base, CH)], idx_v.at[0])
                pltpu.sync_copy(rows_hbm.at[pl.ds(base, CH), pl.ds(off, Dh)],
                                rows_v)
                pltpu.sync_copy(rows_v, acc.at[idx_v.at[0]], add=True)
                return carry

            lax.fori_loop(0, per_w, body, 0)
            plsc.subcore_barrier()
            pltpu.sync_copy(acc.at[pl.ds(seg0, SEG_PER_TILE)],
                            out_hbm.at[pl.ds(seg0, SEG_PER_TILE), pl.ds(off, Dh)])

        pl.when(c == 0)(lambda: run(0))
        pl.when(c == 1)(lambda: run(1))

    return k


@functools.cache
def _sc_scatter_edge(Dh, tiled=True):
    per_w = EP // (NTILES * CH)

    @functools.partial(
        pl.kernel,
        out_type=jax.ShapeDtypeStruct((2 * NSEGP, Dh), _f32),
        mesh=_sc_mesh(),
        scratch_types=[pltpu.VMEM((1, CH), jnp.int32),
                       pltpu.VMEM((CH, Dh), _f32),
                       pltpu.VMEM_SHARED((NSEGP, Dh), _f32)],
        compiler_params=pltpu.CompilerParams(use_tc_tiling_on_sc=tiled),
    )
    def k(rows_hbm, idx_hbm, zer_hbm, out_hbm, idx_v, rows_v, acc):
        c = lax.axis_index("c")
        s = lax.axis_index("s")
        seg0 = pl.multiple_of(s * SEG_PER_TILE, 8)
        pltpu.sync_copy(zer_hbm.at[pl.ds(seg0, SEG_PER_TILE)],
                        acc.at[pl.ds(seg0, SEG_PER_TILE)])
        plsc.subcore_barrier()

        def run(cc):
            def body(j, carry):
                base = pl.multiple_of(((cc * 16 + s) * per_w + j) * CH, CH)
                pltpu.sync_copy(idx_hbm.at[pl.ds(base, CH)], idx_v.at[0])
                pltpu.sync_copy(rows_hbm.at[pl.ds(base, CH)], rows_v)
                pltpu.sync_copy(rows_v, acc.at[idx_v.at[0]], add=True)
                return carry

            lax.fori_loop(0, per_w, body, 0)
            plsc.subcore_barrier()
            ob = pl.multiple_of(cc * NSEGP + seg0, 8)
            pltpu.sync_copy(acc.at[pl.ds(seg0, SEG_PER_TILE)],
                            out_hbm.at[pl.ds(ob, SEG_PER_TILE)])

        pl.when(c == 0)(lambda: run(0))
        pl.when(c == 1)(lambda: run(1))

    return k


def _att_expand(att, D):
    heads, ch = att.shape
    a = jnp.zeros((D, 8), _f32)
    hh = jnp.arange(D) // ch
    return a.at[jnp.arange(D), hh].set(att.reshape(-1))


def _head_expand(heads, ch, D):
    hh = jnp.arange(D) // ch
    return (hh[None, :] == jnp.arange(8)[:, None]).astype(_f32)


def _pack_bf16(x):
    xb = x.astype(jnp.bfloat16)
    return lax.bitcast_convert_type(xb.reshape(x.shape[0], -1, 2), _f32)


def _unpack_bf16(x):
    return lax.bitcast_convert_type(x, jnp.bfloat16).reshape(x.shape[0], -1)


def _gat1(xl, xr, src, dstg, dsts, a1, b1exp):
    xls = _unpack_bf16(_sc_gather(N, 128)(_pack_bf16(xl), src))
    xrd = _unpack_bf16(_sc_gather(N, 128)(_pack_bf16(xr), dstg))
    ee, w = _edge1(xls, xrd, a1, b1exp)
    dpart = _sc_scatter_edge(16, False)(ee, dsts, jnp.zeros((NSEGP, 16), _f32))
    u = _sc_scatter_feat(128)(w, dsts, jnp.zeros((NSEGP, 128), _f32))
    return u, dpart


def _gat4(xl, xr, src, dstg, dsts, a4, b4exp):
    xls = _sc_gather(N, HID, jnp.float32, False)(xl, src)
    xrd = _sc_gather(N, HID, jnp.float32, False)(xr, dstg)
    w = _edge4(xls, xrd, a4, b4exp)
    return _sc_scatter_edge(HID + 8, False)(
        w, dsts, jnp.zeros((NSEGP, HID + 8), _f32))


def kernel(x, edge_index, batch, node_types, ptr, Wt, bt, Wl1, bl1, Wr1, br1,
           att1, bias1, gn1_w, gn1_b, gn1_ms, Wl4, bl4, Wr4, br4, att4, bias4,
           gn4_w, gn4_b, gn4_ms, Wlin, blin):
    loop = jnp.arange(N, dtype=jnp.int32)
    src = jnp.concatenate([edge_index[0], loop,
                           jnp.zeros((PAD,), jnp.int32)])
    dstg = jnp.concatenate([edge_index[1], loop,
                            jnp.zeros((PAD,), jnp.int32)])
    dsts = jnp.concatenate([edge_index[1], loop,
                            N + (jnp.arange(PAD, dtype=jnp.int32) % 16)])

    gids = jnp.arange(B, dtype=jnp.int32)
    M = (batch[None, :] == gids[:, None]).astype(_f32)
    Mt = M.T
    S = (ptr[:-1][:, None] == jnp.arange(N, dtype=jnp.int32)[None, :]).astype(_f32)

    a1 = _att_expand(att1, HEADS * HID)
    b1exp = _head_expand(HEADS, HID, HEADS * HID)
    a4 = _att_expand(att4, HID)
    b4exp = _head_expand(1, HID, HID)

    xl1, xr1 = _dense1(x, Wt, bt, Wl1, bl1, Wr1, br1)
    u1, dp1 = _gat1(xl1, xr1, src, dstg, dsts, a1, b1exp)
    h1, xl4, xr4 = _gn1(u1[:N], dp1[:N], dp1[NSEGP:NSEGP + N], b1exp,
                        bias1, M, Mt, gn1_ms, gn1_w, gn1_b,
                        Wl4, bl4, Wr4, br4)
    u4 = _gat4(xl4, xr4, src, dstg, dsts, a4, b4exp)
    out = _final(u4[:N], u4[NSEGP:NSEGP + N],
                 bias4, M, Mt, gn4_ms, gn4_w, gn4_b, h1, S, Wlin, blin)
    return out.reshape(B)


# speedup vs baseline: 1.8916x; 1.8916x over previous
import functools

import jax
import jax.numpy as jnp
from jax import lax
from jax.experimental import pallas as pl
from jax.experimental.pallas import tpu as pltpu
from jax.experimental.pallas import tpu_sc as plsc

N = 10000
E = 320000
D_IN = 128
HID = 32
HEADS = 8
B = 16

CH = 128
NTILES = 32
EP = 331776
PAD = EP - (E + N)
NSEGP = 10112
SEG_PER_TILE = NSEGP // 16

_f32 = jnp.float32


def _dense1_body(x_ref, wt_ref, bt_ref, wl_ref, bl_ref, wr_ref, br_ref,
                 xl_ref, xr_ref):
    h0 = jnp.dot(x_ref[...], wt_ref[...], preferred_element_type=_f32) + bt_ref[...]
    xl_ref[...] = jnp.dot(h0, wl_ref[...], preferred_element_type=_f32) + bl_ref[...]
    xr_ref[...] = jnp.dot(h0, wr_ref[...], preferred_element_type=_f32) + br_ref[...]


def _dense1(x, Wt, bt, Wl, bl, Wr, br):
    K = Wl.shape[1]
    return pl.pallas_call(
        _dense1_body,
        out_shape=(jax.ShapeDtypeStruct((N, K), _f32),
                   jax.ShapeDtypeStruct((N, K), _f32)),
    )(x, Wt, bt.reshape(1, -1), Wl, bl.reshape(1, -1), Wr, br.reshape(1, -1))


def _unpack_lanes(x):
    xi = lax.bitcast_convert_type(x, jnp.int32)
    lo = lax.bitcast_convert_type(xi << 16, _f32)
    hi = lax.bitcast_convert_type(xi & jnp.int32(-65536), _f32)
    return lo, hi


def _edge1_body(xl_ref, xr_ref, a_ref, b_ref, ee_ref, w_ref):
    xl_lo, xl_hi = _unpack_lanes(xl_ref[...])
    xr_lo, xr_hi = _unpack_lanes(xr_ref[...])
    m = jnp.concatenate([xl_lo + xr_lo, xl_hi + xr_hi], axis=1)
    m = jnp.where(m > 0, m, 0.2 * m)
    ee = jnp.exp(jnp.dot(m, a_ref[...], preferred_element_type=_f32))
    ee_ref[...] = jnp.concatenate(
        [ee, jnp.zeros((ee.shape[0], 8), _f32)], axis=1)
    af = jnp.dot(ee, b_ref[...], preferred_element_type=_f32)
    w_ref[...] = af * jnp.concatenate([xl_lo, xl_hi], axis=1)


def _edge1(xlsp, xrdp, a, bexp):
    blk = 4096
    return pl.pallas_call(
        _edge1_body,
        grid=(EP // blk,),
        in_specs=[pl.BlockSpec((blk, 128), lambda i: (i, 0)),
                  pl.BlockSpec((blk, 128), lambda i: (i, 0)),
                  pl.BlockSpec((256, 8), lambda i: (0, 0)),
                  pl.BlockSpec((8, 256), lambda i: (0, 0))],
        out_specs=(pl.BlockSpec((blk, 16), lambda i: (i, 0)),
                   pl.BlockSpec((blk, 256), lambda i: (i, 0))),
        out_shape=(jax.ShapeDtypeStruct((EP, 16), _f32),
                   jax.ShapeDtypeStruct((EP, 256), _f32)),
    )(xlsp, xrdp, a, bexp)


def _edge4_body(xl_ref, xr_ref, a_ref, b_ref, w_ref):
    m = xl_ref[...] + xr_ref[...]
    m = jnp.where(m > 0, m, 0.2 * m)
    ee = jnp.exp(jnp.dot(m, a_ref[...], preferred_element_type=_f32))
    af = jnp.dot(ee, b_ref[...], preferred_element_type=_f32)
    w = af * xl_ref[...]
    w_ref[...] = jnp.concatenate(
        [w[:, :HID], ee[:, :1],
         jnp.zeros((w.shape[0], 128 - HID - 1), _f32)], axis=1)


def _edge4(xls, xrd, a, bexp):
    blk = 4096
    return pl.pallas_call(
        _edge4_body,
        grid=(EP // blk,),
        in_specs=[pl.BlockSpec((blk, 128), lambda i: (i, 0)),
                  pl.BlockSpec((blk, 128), lambda i: (i, 0)),
                  pl.BlockSpec((128, 8), lambda i: (0, 0)),
                  pl.BlockSpec((8, 128), lambda i: (0, 0))],
        out_specs=pl.BlockSpec((blk, 128), lambda i: (i, 0)),
        out_shape=jax.ShapeDtypeStruct((EP, 128), _f32),
    )(xls, xrd, a, bexp)


def _gn1_body(u_ref, d0_ref, d1_ref, be_ref, b1_ref, m_ref, mt_ref, ms_ref,
              w_ref, bb_ref, wl_ref, bl_ref, wr_ref, br_ref,
              h1_ref, xl_ref, xr_ref):
    den8 = d0_ref[...][:, :8] + d1_ref[...][:, :8]
    den = jnp.dot(den8, be_ref[...], preferred_element_type=_f32)
    x = u_ref[...] / (den + 1e-16) + b1_ref[...]
    M = m_ref[...]
    Mt = mt_ref[...]
    cnt = jnp.maximum(jnp.sum(M, axis=1, keepdims=True), 1.0)
    mean = jnp.dot(M, x, preferred_element_type=_f32) / cnt
    xc = x - jnp.dot(Mt, mean, preferred_element_type=_f32) * ms_ref[...]
    var = jnp.dot(M, xc * xc, preferred_element_type=_f32) / cnt
    rstd = lax.rsqrt(var + 1e-5)
    y = w_ref[...] * xc * jnp.dot(Mt, rstd, preferred_element_type=_f32) + bb_ref[...]
    h = jnp.where(y > 0, y, jnp.exp(jnp.minimum(y, 0.0)) - 1.0)
    h1_ref[...] = h
    pad = jnp.zeros((N, 128 - HID), _f32)
    xl = jnp.dot(h, wl_ref[...], preferred_element_type=_f32) + bl_ref[...]
    xr = jnp.dot(h, wr_ref[...], preferred_element_type=_f32) + br_ref[...]
    xl_ref[...] = jnp.concatenate([xl, pad], axis=1)
    xr_ref[...] = jnp.concatenate([xr, pad], axis=1)


def _gn1(u, d0, d1, bexp, bias1, M, Mt, ms, w, b, Wl4, bl4, Wr4, br4):
    return pl.pallas_call(
        _gn1_body,
        out_shape=(jax.ShapeDtypeStruct((N, HEADS * HID), _f32),
                   jax.ShapeDtypeStruct((N, 128), _f32),
                   jax.ShapeDtypeStruct((N, 128), _f32)),
    )(u, d0, d1, bexp, bias1.reshape(1, -1), M, Mt, ms.reshape(1, -1),
      w.reshape(1, -1), b.reshape(1, -1), Wl4, bl4.reshape(1, -1),
      Wr4, br4.reshape(1, -1))


def _final_body(ua_ref, ub_ref, b4_ref, m_ref, mt_ref,
                ms_ref, w_ref, bb_ref, h1_ref, s_ref, wl1_ref, wl2_ref,
                blin_ref, out_ref):
    upacked = ua_ref[...] + ub_ref[...]
    den1 = upacked[:, HID:HID + 1]
    u = upacked[:, :HID]
    x = u / (den1 + 1e-16) + b4_ref[...]
    M = m_ref[...]
    Mt = mt_ref[...]
    cnt = jnp.maximum(jnp.sum(M, axis=1, keepdims=True), 1.0)
    mean = jnp.dot(M, x, preferred_element_type=_f32) / cnt
    xc = x - jnp.dot(Mt, mean, preferred_element_type=_f32) * ms_ref[...]
    var = jnp.dot(M, xc * xc, preferred_element_type=_f32) / cnt
    rstd = lax.rsqrt(var + 1e-5)
    h2 = w_ref[...] * xc * jnp.dot(Mt, rstd, preferred_element_type=_f32) + bb_ref[...]
    S = s_ref[...]
    s1 = jnp.dot(S, h1_ref[...], preferred_element_type=_f32)
    s2 = jnp.dot(S, h2, preferred_element_type=_f32)
    out = (jnp.sum(s1 * wl1_ref[...], axis=1, keepdims=True)
           + jnp.sum(s2 * wl2_ref[...], axis=1, keepdims=True)
           + blin_ref[...])
    out_ref[...] = out


def _final(u0, u1, bias4, M, Mt, ms, w, b, h1, S, Wlin, blin):
    wl1 = Wlin[:HEADS * HID].reshape(1, -1)
    wl2 = Wlin[HEADS * HID:].reshape(1, -1)
    return pl.pallas_call(
        _final_body,
        out_shape=jax.ShapeDtypeStruct((B, 1), _f32),
    )(u0, u1, bias4.reshape(1, -1), M, Mt, ms.reshape(1, -1),
      w.reshape(1, -1), b.reshape(1, -1), h1, S, wl1, wl2, blin.reshape(1, 1))


@functools.cache
def _sc_mesh():
    return plsc.VectorSubcoreMesh(core_axis_name="c", subcore_axis_name="s")


@functools.cache
def _sc_gather(n_table, D, dtype=jnp.float32, tiled=True):
    per_w = EP // (NTILES * CH)

    @functools.partial(
        pl.kernel,
        out_type=jax.ShapeDtypeStruct((EP, D), dtype),
        mesh=_sc_mesh(),
        scratch_types=[pltpu.VMEM((1, CH), jnp.int32),
                       pltpu.VMEM((CH, D), dtype),
                       pltpu.SemaphoreType.DMA],
        compiler_params=pltpu.CompilerParams(use_tc_tiling_on_sc=tiled),
    )
    def k(table_hbm, idx_hbm, out_hbm, idx_v, rows_v, sem):
        c = lax.axis_index("c")
        s = lax.axis_index("s")
        wid = s * 2 + c

        def body(j, carry):
            base = pl.multiple_of((wid * per_w + j) * CH, CH)
            pltpu.sync_copy(idx_hbm.at[pl.ds(base, CH)], idx_v.at[0])
            pltpu.async_copy(table_hbm.at[idx_v.at[0]], rows_v, sem).wait()
            pltpu.sync_copy(rows_v, out_hbm.at[pl.ds(base, CH)])
            return carry

        lax.fori_loop(0, per_w, body, 0)

    return k


@functools.cache
def _sc_scatter_feat(Dh):
    per_w = EP // (16 * CH)

    @functools.partial(
        pl.kernel,
        out_type=jax.ShapeDtypeStruct((NSEGP, 2 * Dh), _f32),
        mesh=_sc_mesh(),
        scratch_types=[pltpu.VMEM((1, CH), jnp.int32),
                       pltpu.VMEM((CH, Dh), _f32),
                       pltpu.VMEM_SHARED((NSEGP, Dh), _f32)],
    )
    def k(rows_hbm, idx_hbm, zer_hbm, out_hbm, idx_v, rows_v, acc):
        c = lax.axis_index("c")
        s = lax.axis_index("s")
        seg0 = pl.multiple_of(s * SEG_PER_TILE, 8)
        pltpu.sync_copy(zer_hbm.at[pl.ds(seg0, SEG_PER_TILE)],
                        acc.at[pl.ds(seg0, SEG_PER_TILE)])
        plsc.subcore_barrier()

        def run(cc):
            off = cc * Dh

            def body(j, carry):
                base = pl.multiple_of((s * per_w + j) * CH, CH)
                pltpu.sync_copy(idx_hbm.at[pl.ds(base, CH)], idx_v.at[0])
                pltpu.sync_copy(rows_hbm.at[pl.ds(base, CH), pl.ds(off, Dh)],
                                rows_v)
                pltpu.sync_copy(rows_v, acc.at[idx_v.at[0]], add=True)
                return carry

            lax.fori_loop(0, per_w, body, 0)
            plsc.subcore_barrier()
            pltpu.sync_copy(acc.at[pl.ds(seg0, SEG_PER_TILE)],
                            out_hbm.at[pl.ds(seg0, SEG_PER_TILE), pl.ds(off, Dh)])

        pl.when(c == 0)(lambda: run(0))
        pl.when(c == 1)(lambda: run(1))

    return k


@functools.cache
def _sc_scatter_edge(Dh, tiled=True):
    per_w = EP // (NTILES * CH)

    @functools.partial(
        pl.kernel,
        out_type=jax.ShapeDtypeStruct((2 * NSEGP, Dh), _f32),
        mesh=_sc_mesh(),
        scratch_types=[pltpu.VMEM((1, CH), jnp.int32),
                       pltpu.VMEM((CH, Dh), _f32),
                       pltpu.VMEM_SHARED((NSEGP, Dh), _f32)],
        compiler_params=pltpu.CompilerParams(use_tc_tiling_on_sc=tiled),
    )
    def k(rows_hbm, idx_hbm, zer_hbm, out_hbm, idx_v, rows_v, acc):
        c = lax.axis_index("c")
        s = lax.axis_index("s")
        seg0 = pl.multiple_of(s * SEG_PER_TILE, 8)
        pltpu.sync_copy(zer_hbm.at[pl.ds(seg0, SEG_PER_TILE)],
                        acc.at[pl.ds(seg0, SEG_PER_TILE)])
        plsc.subcore_barrier()

        def run(cc):
            def body(j, carry):
                base = pl.multiple_of(((cc * 16 + s) * per_w + j) * CH, CH)
                pltpu.sync_copy(idx_hbm.at[pl.ds(base, CH)], idx_v.at[0])
                pltpu.sync_copy(rows_hbm.at[pl.ds(base, CH)], rows_v)
                pltpu.sync_copy(rows_v, acc.at[idx_v.at[0]], add=True)
                return carry

            lax.fori_loop(0, per_w, body, 0)
            plsc.subcore_barrier()
            ob = pl.multiple_of(cc * NSEGP + seg0, 8)
            pltpu.sync_copy(acc.at[pl.ds(seg0, SEG_PER_TILE)],
                            out_hbm.at[pl.ds(ob, SEG_PER_TILE)])

        pl.when(c == 0)(lambda: run(0))
        pl.when(c == 1)(lambda: run(1))

    return k


def _att_expand(att, D):
    heads, ch = att.shape
    a = jnp.zeros((D, 8), _f32)
    hh = jnp.arange(D) // ch
    return a.at[jnp.arange(D), hh].set(att.reshape(-1))


def _head_expand(heads, ch, D):
    hh = jnp.arange(D) // ch
    return (hh[None, :] == jnp.arange(8)[:, None]).astype(_f32)


def _pack_bf16(x):
    xb = x.astype(jnp.bfloat16)
    return lax.bitcast_convert_type(xb.reshape(x.shape[0], -1, 2), _f32)


def _unpack_bf16(x):
    return lax.bitcast_convert_type(x, jnp.bfloat16).reshape(x.shape[0], -1)


def _gat1(xl, xr, src, dstg, dsts, a1, b1exp):
    xlsp = _sc_gather(N, 128)(_pack_bf16(xl), src)
    xrdp = _sc_gather(N, 128)(_pack_bf16(xr), dstg)
    ee, w = _edge1(xlsp, xrdp, a1, b1exp)
    dpart = _sc_scatter_edge(16, False)(ee, dsts, jnp.zeros((NSEGP, 16), _f32))
    u = _sc_scatter_feat(128)(w, dsts, jnp.zeros((NSEGP, 128), _f32))
    return u, dpart


def _gat4(xl, xr, src, dstg, dsts, a4, b4exp):
    xls = _sc_gather(N, 128)(xl, src)
    xrd = _sc_gather(N, 128)(xr, dstg)
    w = _edge4(xls, xrd, a4, b4exp)
    return _sc_scatter_edge(128)(w, dsts, jnp.zeros((NSEGP, 128), _f32))


def kernel(x, edge_index, batch, node_types, ptr, Wt, bt, Wl1, bl1, Wr1, br1,
           att1, bias1, gn1_w, gn1_b, gn1_ms, Wl4, bl4, Wr4, br4, att4, bias4,
           gn4_w, gn4_b, gn4_ms, Wlin, blin):
    loop = jnp.arange(N, dtype=jnp.int32)
    src = jnp.concatenate([edge_index[0], loop,
                           jnp.zeros((PAD,), jnp.int32)])
    dstg = jnp.concatenate([edge_index[1], loop,
                            jnp.zeros((PAD,), jnp.int32)])
    dsts = jnp.concatenate([edge_index[1], loop,
                            N + (jnp.arange(PAD, dtype=jnp.int32) % 16)])

    gids = jnp.arange(B, dtype=jnp.int32)
    M = (batch[None, :] == gids[:, None]).astype(_f32)
    Mt = M.T
    S = (ptr[:-1][:, None] == jnp.arange(N, dtype=jnp.int32)[None, :]).astype(_f32)

    D1 = HEADS * HID
    perm = jnp.concatenate([jnp.arange(0, D1, 2), jnp.arange(1, D1, 2)])
    a1 = _att_expand(att1, D1)[perm, :]
    b1exp = _head_expand(HEADS, HID, D1)[:, perm]
    a4 = jnp.concatenate([_att_expand(att4, HID),
                          jnp.zeros((128 - HID, 8), _f32)], axis=0)
    b4exp = jnp.concatenate([_head_expand(1, HID, HID),
                             jnp.zeros((8, 128 - HID), _f32)], axis=1)
    Wlin_p = jnp.concatenate([Wlin[:D1][perm], Wlin[D1:]], axis=0)

    xl1, xr1 = _dense1(x, Wt, bt, Wl1, bl1, Wr1, br1)
    u1, dp1 = _gat1(xl1, xr1, src, dstg, dsts, a1, b1exp)
    h1, xl4, xr4 = _gn1(u1[:N], dp1[:N], dp1[NSEGP:NSEGP + N], b1exp,
                        bias1[perm], M, Mt, gn1_ms[perm], gn1_w[perm],
                        gn1_b[perm], Wl4[perm], bl4, Wr4[perm], br4)
    u4 = _gat4(xl4, xr4, src, dstg, dsts, a4, b4exp)
    out = _final(u4[:N], u4[NSEGP:NSEGP + N],
                 bias4, M, Mt, gn4_ms, gn4_w, gn4_b, h1, S, Wlin_p, blin)
    return out.reshape(B)


# speedup vs baseline: 2.1217x; 1.1216x over previous
import functools

import jax
import jax.numpy as jnp
from jax import lax
from jax.experimental import pallas as pl
from jax.experimental.pallas import tpu as pltpu
from jax.experimental.pallas import tpu_sc as plsc

N = 10000
E = 320000
D_IN = 128
HID = 32
HEADS = 8
B = 16

CH = 128
NTILES = 32
EP = 331776
PAD = EP - (E + N)
NSEGP = 10112
SEG_PER_TILE = NSEGP // 16

_f32 = jnp.float32


def _dense1_body(x_ref, wt_ref, bt_ref, wl_ref, bl_ref, wr_ref, br_ref,
                 xl_ref, xr_ref):
    h0 = jnp.dot(x_ref[...], wt_ref[...], preferred_element_type=_f32) + bt_ref[...]
    xl_ref[...] = jnp.dot(h0, wl_ref[...], preferred_element_type=_f32) + bl_ref[...]
    xr_ref[...] = jnp.dot(h0, wr_ref[...], preferred_element_type=_f32) + br_ref[...]


def _dense1(x, Wt, bt, Wl, bl, Wr, br):
    K = Wl.shape[1]
    return pl.pallas_call(
        _dense1_body,
        out_shape=(jax.ShapeDtypeStruct((N, K), _f32),
                   jax.ShapeDtypeStruct((N, K), _f32)),
    )(x, Wt, bt.reshape(1, -1), Wl, bl.reshape(1, -1), Wr, br.reshape(1, -1))


def _unpack_lanes(x):
    xi = lax.bitcast_convert_type(x, jnp.int32)
    lo = lax.bitcast_convert_type(xi << 16, _f32)
    hi = lax.bitcast_convert_type(xi & jnp.int32(-65536), _f32)
    return lo, hi


def _edge1_body(xl_ref, xr_ref, a_ref, b_ref, ee_ref, w_ref):
    xl_lo, xl_hi = _unpack_lanes(xl_ref[...])
    xr_lo, xr_hi = _unpack_lanes(xr_ref[...])
    m = jnp.concatenate([xl_lo + xr_lo, xl_hi + xr_hi], axis=1)
    m = jnp.where(m > 0, m, 0.2 * m)
    ee = jnp.exp(jnp.dot(m, a_ref[...], preferred_element_type=_f32))
    ee_ref[...] = jnp.concatenate(
        [ee, jnp.zeros((ee.shape[0], 8), _f32)], axis=1)
    af = jnp.dot(ee, b_ref[...], preferred_element_type=_f32)
    w_ref[...] = af * jnp.concatenate([xl_lo, xl_hi], axis=1)


def _edge1(xlsp, xrdp, a, bexp):
    blk = 4096
    return pl.pallas_call(
        _edge1_body,
        grid=(EP // blk,),
        in_specs=[pl.BlockSpec((blk, 128), lambda i: (i, 0)),
                  pl.BlockSpec((blk, 128), lambda i: (i, 0)),
                  pl.BlockSpec((256, 8), lambda i: (0, 0)),
                  pl.BlockSpec((8, 256), lambda i: (0, 0))],
        out_specs=(pl.BlockSpec((blk, 16), lambda i: (i, 0)),
                   pl.BlockSpec((blk, 256), lambda i: (i, 0))),
        out_shape=(jax.ShapeDtypeStruct((EP, 16), _f32),
                   jax.ShapeDtypeStruct((EP, 256), _f32)),
    )(xlsp, xrdp, a, bexp)


def _edge4_body(xl_ref, xr_ref, a_ref, b_ref, w_ref):
    m = xl_ref[...] + xr_ref[...]
    m = jnp.where(m > 0, m, 0.2 * m)
    ee = jnp.exp(jnp.dot(m, a_ref[...], preferred_element_type=_f32))
    af = jnp.dot(ee, b_ref[...], preferred_element_type=_f32)
    w = af * xl_ref[...]
    w_ref[...] = jnp.concatenate(
        [w[:, :HID], ee[:, :1],
         jnp.zeros((w.shape[0], 128 - HID - 1), _f32)], axis=1)


def _edge4(xls, xrd, a, bexp):
    blk = 4096
    return pl.pallas_call(
        _edge4_body,
        grid=(EP // blk,),
        in_specs=[pl.BlockSpec((blk, 128), lambda i: (i, 0)),
                  pl.BlockSpec((blk, 128), lambda i: (i, 0)),
                  pl.BlockSpec((128, 8), lambda i: (0, 0)),
                  pl.BlockSpec((8, 128), lambda i: (0, 0))],
        out_specs=pl.BlockSpec((blk, 128), lambda i: (i, 0)),
        out_shape=jax.ShapeDtypeStruct((EP, 128), _f32),
    )(xls, xrd, a, bexp)


def _gn1_body(u_ref, d0_ref, d1_ref, be_ref, b1_ref, m_ref, mt_ref, ms_ref,
              w_ref, bb_ref, wl_ref, bl_ref, wr_ref, br_ref,
              h1_ref, xl_ref, xr_ref):
    den8 = d0_ref[...][:, :8] + d1_ref[...][:, :8]
    den = jnp.dot(den8, be_ref[...], preferred_element_type=_f32)
    x = u_ref[...] / (den + 1e-16) + b1_ref[...]
    M = m_ref[...]
    Mt = mt_ref[...]
    cnt = jnp.maximum(jnp.sum(M, axis=1, keepdims=True), 1.0)
    mean = jnp.dot(M, x, preferred_element_type=_f32) / cnt
    xc = x - jnp.dot(Mt, mean, preferred_element_type=_f32) * ms_ref[...]
    var = jnp.dot(M, xc * xc, preferred_element_type=_f32) / cnt
    rstd = lax.rsqrt(var + 1e-5)
    y = w_ref[...] * xc * jnp.dot(Mt, rstd, preferred_element_type=_f32) + bb_ref[...]
    h = jnp.where(y > 0, y, jnp.exp(jnp.minimum(y, 0.0)) - 1.0)
    h1_ref[...] = h
    pad = jnp.zeros((N, 128 - HID), _f32)
    xl = jnp.dot(h, wl_ref[...], preferred_element_type=_f32) + bl_ref[...]
    xr = jnp.dot(h, wr_ref[...], preferred_element_type=_f32) + br_ref[...]
    xl_ref[...] = jnp.concatenate([xl, pad], axis=1)
    xr_ref[...] = jnp.concatenate([xr, pad], axis=1)


def _gn1(u, d0, d1, bexp, bias1, M, Mt, ms, w, b, Wl4, bl4, Wr4, br4):
    return pl.pallas_call(
        _gn1_body,
        out_shape=(jax.ShapeDtypeStruct((N, HEADS * HID), _f32),
                   jax.ShapeDtypeStruct((N, 128), _f32),
                   jax.ShapeDtypeStruct((N, 128), _f32)),
    )(u, d0, d1, bexp, bias1.reshape(1, -1), M, Mt, ms.reshape(1, -1),
      w.reshape(1, -1), b.reshape(1, -1), Wl4, bl4.reshape(1, -1),
      Wr4, br4.reshape(1, -1))


def _final_body(ua_ref, ub_ref, b4_ref, m_ref, mt_ref,
                ms_ref, w_ref, bb_ref, h1_ref, s_ref, wl1_ref, wl2_ref,
                blin_ref, out_ref):
    upacked = ua_ref[...] + ub_ref[...]
    den1 = upacked[:, HID:HID + 1]
    u = upacked[:, :HID]
    x = u / (den1 + 1e-16) + b4_ref[...]
    M = m_ref[...]
    Mt = mt_ref[...]
    cnt = jnp.maximum(jnp.sum(M, axis=1, keepdims=True), 1.0)
    mean = jnp.dot(M, x, preferred_element_type=_f32) / cnt
    xc = x - jnp.dot(Mt, mean, preferred_element_type=_f32) * ms_ref[...]
    var = jnp.dot(M, xc * xc, preferred_element_type=_f32) / cnt
    rstd = lax.rsqrt(var + 1e-5)
    h2 = w_ref[...] * xc * jnp.dot(Mt, rstd, preferred_element_type=_f32) + bb_ref[...]
    S = s_ref[...]
    s1 = jnp.dot(S, h1_ref[...], preferred_element_type=_f32)
    s2 = jnp.dot(S, h2, preferred_element_type=_f32)
    out = (jnp.sum(s1 * wl1_ref[...], axis=1, keepdims=True)
           + jnp.sum(s2 * wl2_ref[...], axis=1, keepdims=True)
           + blin_ref[...])
    out_ref[...] = out


def _final(u0, u1, bias4, M, Mt, ms, w, b, h1, S, Wlin, blin):
    wl1 = Wlin[:HEADS * HID].reshape(1, -1)
    wl2 = Wlin[HEADS * HID:].reshape(1, -1)
    return pl.pallas_call(
        _final_body,
        out_shape=jax.ShapeDtypeStruct((B, 1), _f32),
    )(u0, u1, bias4.reshape(1, -1), M, Mt, ms.reshape(1, -1),
      w.reshape(1, -1), b.reshape(1, -1), h1, S, wl1, wl2, blin.reshape(1, 1))


@functools.cache
def _sc_mesh():
    return plsc.VectorSubcoreMesh(core_axis_name="c", subcore_axis_name="s")


@functools.cache
def _sc_gather(n_table, D, dtype=jnp.float32, tiled=True, ch=CH):
    per_w = EP // (NTILES * ch)

    @functools.partial(
        pl.kernel,
        out_type=jax.ShapeDtypeStruct((EP, D), dtype),
        mesh=_sc_mesh(),
        scratch_types=[pltpu.VMEM((1, ch), jnp.int32),
                       pltpu.VMEM((ch, D), dtype),
                       pltpu.SemaphoreType.DMA],
        compiler_params=pltpu.CompilerParams(use_tc_tiling_on_sc=tiled),
    )
    def k(table_hbm, idx_hbm, out_hbm, idx_v, rows_v, sem):
        c = lax.axis_index("c")
        s = lax.axis_index("s")
        wid = s * 2 + c

        def body(j, carry):
            base = pl.multiple_of((wid * per_w + j) * ch, ch)
            pltpu.sync_copy(idx_hbm.at[pl.ds(base, ch)], idx_v.at[0])
            pltpu.async_copy(table_hbm.at[idx_v.at[0]], rows_v, sem).wait()
            pltpu.sync_copy(rows_v, out_hbm.at[pl.ds(base, ch)])
            return carry

        lax.fori_loop(0, per_w, body, 0)

    return k


@functools.cache
def _sc_scatter_feat(Dh):
    per_w = EP // (16 * CH)

    @functools.partial(
        pl.kernel,
        out_type=jax.ShapeDtypeStruct((NSEGP, 2 * Dh), _f32),
        mesh=_sc_mesh(),
        scratch_types=[pltpu.VMEM((1, CH), jnp.int32),
                       pltpu.VMEM((CH, Dh), _f32),
                       pltpu.VMEM_SHARED((NSEGP, Dh), _f32)],
    )
    def k(rows_hbm, idx_hbm, zer_hbm, out_hbm, idx_v, rows_v, acc):
        c = lax.axis_index("c")
        s = lax.axis_index("s")
        seg0 = pl.multiple_of(s * SEG_PER_TILE, 8)
        pltpu.sync_copy(zer_hbm.at[pl.ds(seg0, SEG_PER_TILE)],
                        acc.at[pl.ds(seg0, SEG_PER_TILE)])
        plsc.subcore_barrier()

        def run(cc):
            off = cc * Dh

            def body(j, carry):
                base = pl.multiple_of((s * per_w + j) * CH, CH)
                pltpu.sync_copy(idx_hbm.at[pl.ds(base, CH)], idx_v.at[0])
                pltpu.sync_copy(rows_hbm.at[pl.ds(base, CH), pl.ds(off, Dh)],
                                rows_v)
                pltpu.sync_copy(rows_v, acc.at[idx_v.at[0]], add=True)
                return carry

            lax.fori_loop(0, per_w, body, 0)
            plsc.subcore_barrier()
            pltpu.sync_copy(acc.at[pl.ds(seg0, SEG_PER_TILE)],
                            out_hbm.at[pl.ds(seg0, SEG_PER_TILE), pl.ds(off, Dh)])

        pl.when(c == 0)(lambda: run(0))
        pl.when(c == 1)(lambda: run(1))

    return k


@functools.cache
def _sc_scatter_edge(Dh, tiled=True):
    per_w = EP // (NTILES * CH)

    @functools.partial(
        pl.kernel,
        out_type=jax.ShapeDtypeStruct((2 * NSEGP, Dh), _f32),
        mesh=_sc_mesh(),
        scratch_types=[pltpu.VMEM((1, CH), jnp.int32),
                       pltpu.VMEM((CH, Dh), _f32),
                       pltpu.VMEM_SHARED((NSEGP, Dh), _f32)],
        compiler_params=pltpu.CompilerParams(use_tc_tiling_on_sc=tiled),
    )
    def k(rows_hbm, idx_hbm, zer_hbm, out_hbm, idx_v, rows_v, acc):
        c = lax.axis_index("c")
        s = lax.axis_index("s")
        seg0 = pl.multiple_of(s * SEG_PER_TILE, 8)
        pltpu.sync_copy(zer_hbm.at[pl.ds(seg0, SEG_PER_TILE)],
                        acc.at[pl.ds(seg0, SEG_PER_TILE)])
        plsc.subcore_barrier()

        def run(cc):
            def body(j, carry):
                base = pl.multiple_of(((cc * 16 + s) * per_w + j) * CH, CH)
                pltpu.sync_copy(idx_hbm.at[pl.ds(base, CH)], idx_v.at[0])
                pltpu.sync_copy(rows_hbm.at[pl.ds(base, CH)], rows_v)
                pltpu.sync_copy(rows_v, acc.at[idx_v.at[0]], add=True)
                return carry

            lax.fori_loop(0, per_w, body, 0)
            plsc.subcore_barrier()
            ob = pl.multiple_of(cc * NSEGP + seg0, 8)
            pltpu.sync_copy(acc.at[pl.ds(seg0, SEG_PER_TILE)],
                            out_hbm.at[pl.ds(ob, SEG_PER_TILE)])

        pl.when(c == 0)(lambda: run(0))
        pl.when(c == 1)(lambda: run(1))

    return k


def _att_expand(att, D):
    heads, ch = att.shape
    a = jnp.zeros((D, 8), _f32)
    hh = jnp.arange(D) // ch
    return a.at[jnp.arange(D), hh].set(att.reshape(-1))


def _head_expand(heads, ch, D):
    hh = jnp.arange(D) // ch
    return (hh[None, :] == jnp.arange(8)[:, None]).astype(_f32)


def _pack_bf16(x):
    xb = x.astype(jnp.bfloat16)
    return lax.bitcast_convert_type(xb.reshape(x.shape[0], -1, 2), _f32)


def _unpack_bf16(x):
    return lax.bitcast_convert_type(x, jnp.bfloat16).reshape(x.shape[0], -1)


def _gat1(xl, xr, src, dstg, dsts, a1, b1exp):
    xlsp = _sc_gather(N, 128, ch=648)(_pack_bf16(xl), src)
    xrdp = _sc_gather(N, 128, ch=648)(_pack_bf16(xr), dstg)
    ee, w = _edge1(xlsp, xrdp, a1, b1exp)
    dpart = _sc_scatter_edge(16, False)(ee, dsts, jnp.zeros((NSEGP, 16), _f32))
    u = _sc_scatter_feat(128)(w, dsts, jnp.zeros((NSEGP, 128), _f32))
    return u, dpart


def _gat4(xl, xr, src, dstg, dsts, a4, b4exp):
    xls = _sc_gather(N, 128, ch=648)(xl, src)
    xrd = _sc_gather(N, 128, ch=648)(xr, dstg)
    w = _edge4(xls, xrd, a4, b4exp)
    return _sc_scatter_edge(128)(w, dsts, jnp.zeros((NSEGP, 128), _f32))


def kernel(x, edge_index, batch, node_types, ptr, Wt, bt, Wl1, bl1, Wr1, br1,
           att1, bias1, gn1_w, gn1_b, gn1_ms, Wl4, bl4, Wr4, br4, att4, bias4,
           gn4_w, gn4_b, gn4_ms, Wlin, blin):
    loop = jnp.arange(N, dtype=jnp.int32)
    src = jnp.concatenate([edge_index[0], loop,
                           jnp.zeros((PAD,), jnp.int32)])
    dstg = jnp.concatenate([edge_index[1], loop,
                            jnp.zeros((PAD,), jnp.int32)])
    dsts = jnp.concatenate([edge_index[1], loop,
                            N + (jnp.arange(PAD, dtype=jnp.int32) % 16)])

    gids = jnp.arange(B, dtype=jnp.int32)
    M = (batch[None, :] == gids[:, None]).astype(_f32)
    Mt = M.T
    S = (ptr[:-1][:, None] == jnp.arange(N, dtype=jnp.int32)[None, :]).astype(_f32)

    D1 = HEADS * HID
    perm = jnp.concatenate([jnp.arange(0, D1, 2), jnp.arange(1, D1, 2)])
    a1 = _att_expand(att1, D1)[perm, :]
    b1exp = _head_expand(HEADS, HID, D1)[:, perm]
    a4 = jnp.concatenate([_att_expand(att4, HID),
                          jnp.zeros((128 - HID, 8), _f32)], axis=0)
    b4exp = jnp.concatenate([_head_expand(1, HID, HID),
                             jnp.zeros((8, 128 - HID), _f32)], axis=1)
    Wlin_p = jnp.concatenate([Wlin[:D1][perm], Wlin[D1:]], axis=0)

    xl1, xr1 = _dense1(x, Wt, bt, Wl1, bl1, Wr1, br1)
    u1, dp1 = _gat1(xl1, xr1, src, dstg, dsts, a1, b1exp)
    h1, xl4, xr4 = _gn1(u1[:N], dp1[:N], dp1[NSEGP:NSEGP + N], b1exp,
                        bias1[perm], M, Mt, gn1_ms[perm], gn1_w[perm],
                        gn1_b[perm], Wl4[perm], bl4, Wr4[perm], br4)
    u4 = _gat4(xl4, xr4, src, dstg, dsts, a4, b4exp)
    out = _final(u4[:N], u4[NSEGP:NSEGP + N],
                 bias4, M, Mt, gn4_ms, gn4_w, gn4_b, h1, S, Wlin_p, blin)
    return out.reshape(B)
